# Initial kernel scaffold; baseline (speedup 1.0000x reference)
import jax
import jax.numpy as jnp
from jax.experimental import pallas as pl


def kernel(all_community_embeddings, valid_nodes, index, index1, weight, lin1_w, lin1_b, lin2_w, lin2_b):
    raise NotImplementedError("write your pallas kernel here")


# speedup vs baseline: 5.5623x; 5.5623x over previous
import functools

import jax
import jax.numpy as jnp
from jax import lax
from jax.experimental import pallas as pl
from jax.experimental.pallas import tpu as pltpu
from jax.experimental.pallas import tpu_sc as plsc

N_NODES = 10000
N_PAD = 10240
N_EDGES = 320000
D_IN = 128
D_PAD = 144

NC = 2
NS = 16
NW = NC * NS
E_PER_W = N_EDGES // NW
CHUNK = 80
N_CHUNKS = E_PER_W // CHUNK
ROWS_PER_S = N_PAD // NS


def _sc_aggregate(xpad, index, index1, zeros):
    mesh = plsc.VectorSubcoreMesh(core_axis_name="c", subcore_axis_name="s")

    @functools.partial(
        pl.kernel,
        mesh=mesh,
        out_type=jax.ShapeDtypeStruct((NC * N_PAD, D_PAD), jnp.float32),
        scratch_types=[
            pltpu.VMEM_SHARED((N_PAD, D_PAD), jnp.float32),
            pltpu.VMEM((CHUNK,), jnp.int32),
            pltpu.VMEM((CHUNK,), jnp.int32),
            pltpu.VMEM((CHUNK, D_PAD), jnp.float32),
            pltpu.SemaphoreType.DMA,
        ],
        compiler_params=pltpu.CompilerParams(use_tc_tiling_on_sc=False),
    )
    def k(xpad_h, idx_h, idx1_h, zero_h, out_h, acc, idx_v, idx1_v, rows_v, sem):
        c = lax.axis_index("c")
        s = lax.axis_index("s")
        wid = c * NS + s

        pltpu.sync_copy(zero_h, acc.at[pl.ds(s * ROWS_PER_S, ROWS_PER_S)])
        plsc.subcore_barrier()

        ebase = wid * E_PER_W

        def chunk_body(i, carry):
            off = ebase + i * CHUNK
            pltpu.sync_copy(idx_h.at[pl.ds(off, CHUNK)], idx_v)
            pltpu.sync_copy(idx1_h.at[pl.ds(off, CHUNK)], idx1_v)
            pltpu.async_copy(xpad_h.at[idx_v], rows_v, sem).wait()
            pltpu.sync_copy(rows_v, acc.at[idx1_v], add=True)
            return carry

        lax.fori_loop(0, N_CHUNKS, chunk_body, 0)
        plsc.subcore_barrier()

        obase = c * N_PAD + s * ROWS_PER_S
        pltpu.sync_copy(acc.at[pl.ds(s * ROWS_PER_S, ROWS_PER_S)],
                        out_h.at[pl.ds(obase, ROWS_PER_S)])

    return k(xpad, index, index1, zeros)


_TC_R = 1000


def _tc_body(x_ref, p0_ref, p1_ref, w_ref, w1_ref, b1_ref, w2_ref, b2_ref, o_ref):
    x = x_ref[...]
    p = p0_ref[0] + p1_ref[0]
    aggr_x = p[:, :D_IN]
    deg = p[:, D_IN:D_IN + 1]
    lin1 = jnp.dot(x, w1_ref[...], preferred_element_type=jnp.float32) + b1_ref[...]
    lin2 = jnp.dot(x, w2_ref[...], preferred_element_type=jnp.float32) + b2_ref[...]
    aggr = jnp.dot(aggr_x, w_ref[...], preferred_element_type=jnp.float32)
    o_ref[...] = deg * lin1 + aggr + lin2


def _tc_finish(x, partial, weight, lin1_w, lin1_b, lin2_w, lin2_b):
    grid = N_NODES // _TC_R
    return pl.pallas_call(
        _tc_body,
        grid=(grid,),
        in_specs=[
            pl.BlockSpec((_TC_R, D_IN), lambda i: (i, 0)),
            pl.BlockSpec((1, _TC_R, D_PAD), lambda i: (0, i, 0)),
            pl.BlockSpec((1, _TC_R, D_PAD), lambda i: (1, i, 0)),
            pl.BlockSpec((D_IN, D_IN), lambda i: (0, 0)),
            pl.BlockSpec((D_IN, D_IN), lambda i: (0, 0)),
            pl.BlockSpec((1, D_IN), lambda i: (0, 0)),
            pl.BlockSpec((D_IN, D_IN), lambda i: (0, 0)),
            pl.BlockSpec((1, D_IN), lambda i: (0, 0)),
        ],
        out_specs=pl.BlockSpec((_TC_R, D_IN), lambda i: (i, 0)),
        out_shape=jax.ShapeDtypeStruct((N_NODES, D_IN), jnp.float32),
    )(x, partial, partial, weight, lin1_w, lin1_b, lin2_w, lin2_b)


def kernel(all_community_embeddings, valid_nodes, index, index1, weight,
           lin1_w, lin1_b, lin2_w, lin2_b):
    x = all_community_embeddings.astype(jnp.float32)
    idx = index.astype(jnp.int32)
    idx1 = index1.astype(jnp.int32)

    pad = jnp.zeros((N_NODES, D_PAD - D_IN), jnp.float32).at[:, 0].set(1.0)
    xpad = jnp.concatenate([x, pad], axis=1)
    zeros = jnp.zeros((ROWS_PER_S, D_PAD), jnp.float32)

    partial = _sc_aggregate(xpad, idx, idx1, zeros)
    partial = partial.reshape(NC, N_PAD, D_PAD)
    return _tc_finish(x, partial,
                      weight.astype(jnp.float32),
                      lin1_w.astype(jnp.float32),
                      lin1_b.astype(jnp.float32).reshape(1, D_IN),
                      lin2_w.astype(jnp.float32),
                      lin2_b.astype(jnp.float32).reshape(1, D_IN))


# speedup vs baseline: 9.4642x; 1.7015x over previous
import functools

import jax
import jax.numpy as jnp
from jax import lax
from jax.experimental import pallas as pl
from jax.experimental.pallas import tpu as pltpu
from jax.experimental.pallas import tpu_sc as plsc

N_NODES = 10000
N_PAD = 10240
N_EDGES = 320000
D_IN = 128
D_PAD = 144

NC = 2
NS = 16
NW = NC * NS
E_PER_W = N_EDGES // NW
CHUNK = 128
N_FULL = E_PER_W // CHUNK
TAIL = E_PER_W - N_FULL * CHUNK
N_PAIRS = N_FULL // 2
ROWS_PER_S = N_PAD // NS


def _sc_aggregate(xpad, index, index1, zeros):
    mesh = plsc.VectorSubcoreMesh(core_axis_name="c", subcore_axis_name="s")

    @functools.partial(
        pl.kernel,
        mesh=mesh,
        out_type=jax.ShapeDtypeStruct((NC * N_PAD, D_PAD), jnp.float32),
        scratch_types=[
            pltpu.VMEM_SHARED((N_PAD, D_PAD), jnp.float32),
            pltpu.VMEM((2, CHUNK), jnp.int32),
            pltpu.VMEM((2, CHUNK), jnp.int32),
            pltpu.VMEM((2, CHUNK, D_PAD), jnp.float32),
            pltpu.VMEM((TAIL,), jnp.int32),
            pltpu.VMEM((TAIL,), jnp.int32),
            pltpu.SemaphoreType.DMA,
            pltpu.SemaphoreType.DMA,
        ],
        compiler_params=pltpu.CompilerParams(use_tc_tiling_on_sc=False),
    )
    def k(xpad_h, idx_h, idx1_h, zero_h, out_h,
          acc, idxg, idxs, rows, idxg_t, idxs_t, gs0, gs1):
        c = lax.axis_index("c")
        s = lax.axis_index("s")
        wid = c * NS + s

        pltpu.sync_copy(zero_h, acc.at[pl.ds(s * ROWS_PER_S, ROWS_PER_S)])
        plsc.subcore_barrier()

        ebase = wid * E_PER_W

        def loads(ch, b):
            off = ebase + ch * CHUNK
            pltpu.sync_copy(idx_h.at[pl.ds(off, CHUNK)], idxg.at[b])
            pltpu.sync_copy(idx1_h.at[pl.ds(off, CHUNK)], idxs.at[b])

        def gather_start(b, sem):
            pltpu.async_copy(xpad_h.at[idxg.at[b]], rows.at[b], sem)

        def gather_wait(b, sem):
            pltpu.make_async_copy(xpad_h.at[idxg.at[b]], rows.at[b], sem).wait()

        def scatter(b):
            pltpu.sync_copy(rows.at[b], acc.at[idxs.at[b]], add=True)

        loads(0, 0)
        gather_start(0, gs0)

        def pair_body(i, carry):
            c0 = 2 * i
            loads(c0 + 1, 1)
            gather_start(1, gs1)
            gather_wait(0, gs0)
            scatter(0)

            @pl.when(c0 + 2 < N_FULL)
            def _():
                loads(c0 + 2, 0)
                gather_start(0, gs0)

            gather_wait(1, gs1)
            scatter(1)
            return carry

        lax.fori_loop(0, N_PAIRS, pair_body, 0)

        toff = ebase + N_FULL * CHUNK
        pltpu.sync_copy(idx_h.at[pl.ds(toff, TAIL)], idxg_t)
        pltpu.sync_copy(idx1_h.at[pl.ds(toff, TAIL)], idxs_t)
        pltpu.async_copy(xpad_h.at[idxg_t], rows.at[0, pl.ds(0, TAIL)], gs0).wait()
        pltpu.sync_copy(rows.at[0, pl.ds(0, TAIL)], acc.at[idxs_t], add=True)
        plsc.subcore_barrier()

        obase = c * N_PAD + s * ROWS_PER_S
        pltpu.sync_copy(acc.at[pl.ds(s * ROWS_PER_S, ROWS_PER_S)],
                        out_h.at[pl.ds(obase, ROWS_PER_S)])

    return k(xpad, index, index1, zeros)


_TC_R = 1000


def _tc_body(x_ref, p0_ref, p1_ref, w_ref, w1_ref, b1_ref, w2_ref, b2_ref, o_ref):
    x = x_ref[...]
    p = p0_ref[0] + p1_ref[0]
    aggr_x = p[:, :D_IN]
    deg = p[:, D_IN:D_IN + 1]
    lin1 = jnp.dot(x, w1_ref[...], preferred_element_type=jnp.float32) + b1_ref[...]
    lin2 = jnp.dot(x, w2_ref[...], preferred_element_type=jnp.float32) + b2_ref[...]
    aggr = jnp.dot(aggr_x, w_ref[...], preferred_element_type=jnp.float32)
    o_ref[...] = deg * lin1 + aggr + lin2


def _tc_finish(x, partial, weight, lin1_w, lin1_b, lin2_w, lin2_b):
    grid = N_NODES // _TC_R
    return pl.pallas_call(
        _tc_body,
        grid=(grid,),
        in_specs=[
            pl.BlockSpec((_TC_R, D_IN), lambda i: (i, 0)),
            pl.BlockSpec((1, _TC_R, D_PAD), lambda i: (0, i, 0)),
            pl.BlockSpec((1, _TC_R, D_PAD), lambda i: (1, i, 0)),
            pl.BlockSpec((D_IN, D_IN), lambda i: (0, 0)),
            pl.BlockSpec((D_IN, D_IN), lambda i: (0, 0)),
            pl.BlockSpec((1, D_IN), lambda i: (0, 0)),
            pl.BlockSpec((D_IN, D_IN), lambda i: (0, 0)),
            pl.BlockSpec((1, D_IN), lambda i: (0, 0)),
        ],
        out_specs=pl.BlockSpec((_TC_R, D_IN), lambda i: (i, 0)),
        out_shape=jax.ShapeDtypeStruct((N_NODES, D_IN), jnp.float32),
    )(x, partial, partial, weight, lin1_w, lin1_b, lin2_w, lin2_b)


def kernel(all_community_embeddings, valid_nodes, index, index1, weight,
           lin1_w, lin1_b, lin2_w, lin2_b):
    x = all_community_embeddings.astype(jnp.float32)
    idx = index.astype(jnp.int32)
    idx1 = index1.astype(jnp.int32)

    pad = jnp.zeros((N_NODES, D_PAD - D_IN), jnp.float32).at[:, 0].set(1.0)
    xpad = jnp.concatenate([x, pad], axis=1)
    zeros = jnp.zeros((ROWS_PER_S, D_PAD), jnp.float32)

    partial = _sc_aggregate(xpad, idx, idx1, zeros)
    partial = partial.reshape(NC, N_PAD, D_PAD)
    return _tc_finish(x, partial,
                      weight.astype(jnp.float32),
                      lin1_w.astype(jnp.float32),
                      lin1_b.astype(jnp.float32).reshape(1, D_IN),
                      lin2_w.astype(jnp.float32),
                      lin2_b.astype(jnp.float32).reshape(1, D_IN))
